# baseline (speedup 1.0000x reference)
import functools

import jax
import jax.numpy as jnp
from jax.experimental import pallas as pl
from jax.experimental.pallas import tpu as pltpu

_SMOOTH = 1e-05


def _dice_body(t_ref, p_ref, out_ref, acc_i, acc_u, acc_d):
    b = pl.program_id(0)
    ni = pl.program_id(1)
    nb = pl.num_programs(1)
    nbatch = pl.num_programs(0)

    p = p_ref[0]
    t = t_ref[0]
    C = p.shape[0]
    cls = jax.lax.broadcasted_iota(jnp.int32, (C, 1), 0)
    mask = t == cls
    inter = jnp.sum(jnp.where(mask, p, 0.0), axis=1, keepdims=True)
    sq = jnp.sum(p * p, axis=1, keepdims=True)
    cnt = jnp.sum(jnp.where(mask, 1.0, 0.0), axis=1, keepdims=True)

    @pl.when(jnp.logical_and(b == 0, ni == 0))
    def _():
        acc_d[0] = 0.0

    @pl.when(ni == 0)
    def _():
        acc_i[...] = jnp.zeros_like(acc_i)
        acc_u[...] = jnp.zeros_like(acc_u)

    acc_i[...] += inter
    acc_u[...] += sq + cnt

    @pl.when(ni == nb - 1)
    def _():
        dice = (2.0 * acc_i[...] + _SMOOTH) / (acc_u[...] + _SMOOTH)
        acc_d[0] += jnp.sum(dice)

    @pl.when(jnp.logical_and(b == nbatch - 1, ni == nb - 1))
    def _():
        out_ref[...] = jnp.full((1, 1), 1.0 - acc_d[0] / (nbatch * C),
                                jnp.float32)


@functools.partial(jax.jit, static_argnames=("interpret",))
def _dice_loss(predict, target, interpret=False):
    B, C, N = predict.shape
    t32 = target.astype(jnp.int32)
    NB = 4096
    out = pl.pallas_call(
        _dice_body,
        grid=(B, N // NB),
        in_specs=[
            pl.BlockSpec((1, 1, NB), lambda b, n: (b, 0, n)),
            pl.BlockSpec((1, C, NB), lambda b, n: (b, 0, n)),
        ],
        out_specs=pl.BlockSpec((1, 1), lambda b, n: (0, 0)),
        out_shape=jax.ShapeDtypeStruct((1, 1), jnp.float32),
        scratch_shapes=[
            pltpu.VMEM((C, 1), jnp.float32),
            pltpu.VMEM((C, 1), jnp.float32),
            pltpu.SMEM((1,), jnp.float32),
        ],
        interpret=interpret,
    )(t32, predict)
    return out[0, 0]


def kernel(predict, target):
    return _dice_loss(predict, target)


# speedup vs baseline: 1.0213x; 1.0213x over previous
import functools

import jax
import jax.numpy as jnp
from jax.experimental import pallas as pl
from jax.experimental.pallas import tpu as pltpu

_SMOOTH = 1e-05


def _dice_body(t_ref, p_ref, out_ref, acc_i, acc_u, acc_d):
    b = pl.program_id(0)
    ni = pl.program_id(1)
    nb = pl.num_programs(1)
    nbatch = pl.num_programs(0)

    p = p_ref[0]
    t = t_ref[0]
    C = p.shape[0]
    cls = jax.lax.broadcasted_iota(jnp.int32, (C, 1), 0)
    mask = t == cls

    @pl.when(jnp.logical_and(b == 0, ni == 0))
    def _():
        acc_d[0] = 0.0

    @pl.when(ni == 0)
    def _():
        acc_i[...] = jnp.zeros_like(acc_i)
        acc_u[...] = jnp.zeros_like(acc_u)

    acc_i[...] += jnp.where(mask, p, 0.0)
    acc_u[...] += p * p + jnp.where(mask, 1.0, 0.0)

    @pl.when(ni == nb - 1)
    def _():
        isum = jnp.sum(acc_i[...], axis=1, keepdims=True)
        usum = jnp.sum(acc_u[...], axis=1, keepdims=True)
        dice = (2.0 * isum + _SMOOTH) / (usum + _SMOOTH)
        acc_d[0] += jnp.sum(dice)

    @pl.when(jnp.logical_and(b == nbatch - 1, ni == nb - 1))
    def _():
        out_ref[...] = jnp.full((1, 1), 1.0 - acc_d[0] / (nbatch * C),
                                jnp.float32)


@functools.partial(jax.jit, static_argnames=("interpret",))
def _dice_loss(predict, target, interpret=False):
    B, C, N = predict.shape
    t32 = target.astype(jnp.int32)
    NB = 4096
    out = pl.pallas_call(
        _dice_body,
        grid=(B, N // NB),
        in_specs=[
            pl.BlockSpec((1, 1, NB), lambda b, n: (b, 0, n)),
            pl.BlockSpec((1, C, NB), lambda b, n: (b, 0, n)),
        ],
        out_specs=pl.BlockSpec((1, 1), lambda b, n: (0, 0)),
        out_shape=jax.ShapeDtypeStruct((1, 1), jnp.float32),
        scratch_shapes=[
            pltpu.VMEM((C, NB), jnp.float32),
            pltpu.VMEM((C, NB), jnp.float32),
            pltpu.SMEM((1,), jnp.float32),
        ],
        interpret=interpret,
    )(t32, predict)
    return out[0, 0]


def kernel(predict, target):
    return _dice_loss(predict, target)


# speedup vs baseline: 1.7461x; 1.7097x over previous
import functools

import jax
import jax.numpy as jnp
from jax.experimental import pallas as pl
from jax.experimental.pallas import tpu as pltpu

_SMOOTH = 1e-05


def _dice_body(t_ref, p_ref, out_ref, acc_i, acc_u, acc_d):
    b = pl.program_id(0)
    ni = pl.program_id(1)
    nb = pl.num_programs(1)
    nbatch = pl.num_programs(0)

    p = p_ref[0]
    t = t_ref[0]
    C = p.shape[0]
    cls = jax.lax.broadcasted_iota(jnp.int32, (C, 1), 0)
    mask = t == cls

    @pl.when(jnp.logical_and(b == 0, ni == 0))
    def _():
        acc_d[0] = 0.0

    @pl.when(ni == 0)
    def _():
        acc_i[...] = jnp.zeros_like(acc_i)
        acc_u[...] = jnp.zeros_like(acc_u)

    acc_i[...] += jnp.where(mask, p, 0.0)
    acc_u[...] += p * p + jnp.where(mask, 1.0, 0.0)

    @pl.when(ni == nb - 1)
    def _():
        isum = jnp.sum(acc_i[...], axis=1, keepdims=True)
        usum = jnp.sum(acc_u[...], axis=1, keepdims=True)
        dice = (2.0 * isum + _SMOOTH) / (usum + _SMOOTH)
        acc_d[0] += jnp.sum(dice)

    @pl.when(jnp.logical_and(b == nbatch - 1, ni == nb - 1))
    def _():
        out_ref[...] = jnp.full((1, 1), 1.0 - acc_d[0] / (nbatch * C),
                                jnp.float32)


@functools.partial(jax.jit, static_argnames=("interpret",))
def _dice_loss(predict, target, interpret=False):
    B, C, N = predict.shape
    t32 = target.astype(jnp.int32)
    NB = 16384
    out = pl.pallas_call(
        _dice_body,
        grid=(B, N // NB),
        in_specs=[
            pl.BlockSpec((1, 1, NB), lambda b, n: (b, 0, n)),
            pl.BlockSpec((1, C, NB), lambda b, n: (b, 0, n)),
        ],
        out_specs=pl.BlockSpec((1, 1), lambda b, n: (0, 0)),
        out_shape=jax.ShapeDtypeStruct((1, 1), jnp.float32),
        scratch_shapes=[
            pltpu.VMEM((C, NB), jnp.float32),
            pltpu.VMEM((C, NB), jnp.float32),
            pltpu.SMEM((1,), jnp.float32),
        ],
        interpret=interpret,
    )(t32, predict)
    return out[0, 0]


def kernel(predict, target):
    return _dice_loss(predict, target)


# speedup vs baseline: 1.9513x; 1.1175x over previous
import functools

import jax
import jax.numpy as jnp
from jax.experimental import pallas as pl
from jax.experimental.pallas import tpu as pltpu

_SMOOTH = 1e-05


def _dice_body(t_ref, p_ref, out_ref, acc_i, acc_u, acc_d):
    b = pl.program_id(0)
    ni = pl.program_id(1)
    nb = pl.num_programs(1)
    nbatch = pl.num_programs(0)

    p = p_ref[0]
    t = t_ref[0]
    C = p.shape[0]
    cls = jax.lax.broadcasted_iota(jnp.int32, (C, 1), 0)
    mask = t == cls

    @pl.when(jnp.logical_and(b == 0, ni == 0))
    def _():
        acc_d[0] = 0.0

    @pl.when(ni == 0)
    def _():
        acc_i[...] = jnp.zeros_like(acc_i)
        acc_u[...] = jnp.zeros_like(acc_u)

    acc_i[...] += jnp.where(mask, p, 0.0)
    acc_u[...] += p * p + jnp.where(mask, 1.0, 0.0)

    @pl.when(ni == nb - 1)
    def _():
        isum = jnp.sum(acc_i[...], axis=1, keepdims=True)
        usum = jnp.sum(acc_u[...], axis=1, keepdims=True)
        dice = (2.0 * isum + _SMOOTH) / (usum + _SMOOTH)
        acc_d[0] += jnp.sum(dice)

    @pl.when(jnp.logical_and(b == nbatch - 1, ni == nb - 1))
    def _():
        out_ref[...] = jnp.full((1, 1), 1.0 - acc_d[0] / (nbatch * C),
                                jnp.float32)


@functools.partial(jax.jit, static_argnames=("interpret",))
def _dice_loss(predict, target, interpret=False):
    B, C, N = predict.shape
    t32 = target.astype(jnp.int32)
    NB = 32768
    out = pl.pallas_call(
        _dice_body,
        grid=(B, N // NB),
        in_specs=[
            pl.BlockSpec((1, 1, NB), lambda b, n: (b, 0, n)),
            pl.BlockSpec((1, C, NB), lambda b, n: (b, 0, n)),
        ],
        out_specs=pl.BlockSpec((1, 1), lambda b, n: (0, 0)),
        out_shape=jax.ShapeDtypeStruct((1, 1), jnp.float32),
        scratch_shapes=[
            pltpu.VMEM((C, NB), jnp.float32),
            pltpu.VMEM((C, NB), jnp.float32),
            pltpu.SMEM((1,), jnp.float32),
        ],
        interpret=interpret,
    )(t32, predict)
    return out[0, 0]


def kernel(predict, target):
    return _dice_loss(predict, target)
